# baseline (device time: 12841 ns/iter reference)
import jax
import jax.numpy as jnp
from jax import lax
from jax.experimental import pallas as pl
from jax.experimental.pallas import tpu as pltpu

N_DEV = 16

_SCORE_DOT = (((2,), (1,)), ((0,), (0,)))
_CTX_DOT = (((2,), (2,)), ((0,), (0,)))


def kernel(x, Wq, K_ext, V_ext, Wo):
    b, sq_loc, d_model = x.shape
    _, _, hq, dh = K_ext.shape

    kt = jnp.transpose(K_ext, (0, 2, 3, 1)).astype(jnp.bfloat16)
    vt = jnp.transpose(V_ext, (0, 2, 3, 1)).astype(jnp.bfloat16)
    xb = x.astype(jnp.bfloat16)
    wqb = Wq.astype(jnp.bfloat16)
    wob = Wo.astype(jnp.bfloat16)

    def body(x_hbm, wq_hbm, kt_hbm, vt_hbm, wo_hbm, out_hbm,
             x_v, wq_v, kt_v, vt_v, wo_v, out_v, kv_win,
             in_sems, out_sem, send_sems, recv_sems):
        my = lax.axis_index("i")
        left = lax.rem(my - 1 + N_DEV, N_DEV)
        right = lax.rem(my + 1, N_DEV)
        has_left = my != 0
        has_right = my != N_DEV - 1

        in_cps = [
            pltpu.make_async_copy(src, dst, in_sems.at[i])
            for i, (src, dst) in enumerate([
                (kt_hbm, kt_v), (vt_hbm, vt_v),
                (x_hbm, x_v), (wq_hbm, wq_v), (wo_hbm, wo_v),
            ])
        ]
        for cp in in_cps:
            cp.start()

        barrier_sem = pltpu.get_barrier_semaphore()

        @pl.when(has_right)
        def _():
            pl.semaphore_signal(
                barrier_sem, inc=1,
                device_id=(right,), device_id_type=pl.DeviceIdType.MESH,
            )

        @pl.when(has_left)
        def _():
            pl.semaphore_signal(
                barrier_sem, inc=1,
                device_id=(left,), device_id_type=pl.DeviceIdType.MESH,
            )

        is_edge = jnp.logical_or(my == 0, my == N_DEV - 1)

        @pl.when(is_edge)
        def _():
            pl.semaphore_wait(barrier_sem, 1)

        @pl.when(jnp.logical_not(is_edge))
        def _():
            pl.semaphore_wait(barrier_sem, 2)

        in_cps[0].wait()
        in_cps[1].wait()
        rdmas = []
        for i, (src, slot, kv, tgt, gate) in enumerate([
            (kt_v, 0, 0, right, has_right),
            (vt_v, 0, 1, right, has_right),
            (kt_v, 1, 0, left, has_left),
            (vt_v, 1, 1, left, has_left),
        ]):
            c = pltpu.make_async_remote_copy(
                src_ref=src, dst_ref=kv_win.at[slot, kv],
                send_sem=send_sems.at[i], recv_sem=recv_sems.at[i],
                device_id=(tgt,), device_id_type=pl.DeviceIdType.MESH,
            )

            @pl.when(gate)
            def _(c=c):
                c.start()

            rdmas.append(c)

        in_cps[2].wait()
        in_cps[3].wait()
        xf = x_v[...].reshape(b * sq_loc, d_model)
        qp = jnp.dot(xf, wq_v[...], preferred_element_type=jnp.float32)
        q = (qp.reshape(b, sq_loc, hq, dh)
             .transpose(0, 2, 1, 3).astype(jnp.bfloat16))

        ctx_acc, den_acc = [], []
        for bb in range(b):
            s = lax.dot_general(q[bb], kt_v[bb], _SCORE_DOT,
                                preferred_element_type=jnp.float32)
            w = jnp.exp(s * 0.125)
            den_acc.append(jnp.sum(w, axis=-1, keepdims=True))
            ctx_acc.append(lax.dot_general(
                w.astype(jnp.bfloat16), vt_v[bb], _CTX_DOT,
                preferred_element_type=jnp.float32))

        qi = lax.broadcasted_iota(jnp.int32, (1, sq_loc, sq_loc), 1)
        kj = lax.broadcasted_iota(jnp.int32, (1, sq_loc, sq_loc), 2)
        for slot, gate, mask in ((0, has_left, qi <= kj),
                                 (1, has_right, kj <= qi)):

            @pl.when(gate)
            def _(slot=slot):
                rdmas[2 * slot].wait_recv()

            ws = []
            for bb in range(b):
                s = lax.dot_general(q[bb], kv_win[slot, 0, bb], _SCORE_DOT,
                                    preferred_element_type=jnp.float32)
                w = jnp.where(mask, jnp.exp(s * 0.125), 0.0)
                den_acc[bb] += jnp.where(gate, jnp.sum(w, axis=-1,
                                                       keepdims=True), 0.0)
                ws.append(w)

            @pl.when(gate)
            def _(slot=slot):
                rdmas[2 * slot + 1].wait_recv()

            for bb in range(b):
                c_add = lax.dot_general(
                    ws[bb].astype(jnp.bfloat16), kv_win[slot, 1, bb],
                    _CTX_DOT, preferred_element_type=jnp.float32)
                ctx_acc[bb] += jnp.where(gate, c_add, 0.0)

        ctx = jnp.stack([
            (ctx_acc[bb] / den_acc[bb]).transpose(1, 0, 2)
            for bb in range(b)
        ]).reshape(b * sq_loc, hq * dh)

        in_cps[4].wait()
        o = jnp.dot(ctx.astype(jnp.bfloat16), wo_v[...],
                    preferred_element_type=jnp.float32)
        out_v[...] = o.reshape(b, sq_loc, d_model).astype(jnp.bfloat16)

        out_cp = pltpu.make_async_copy(out_v, out_hbm, out_sem)
        out_cp.start()
        out_cp.wait()

        for i, gate in enumerate([has_right, has_right, has_left, has_left]):

            @pl.when(gate)
            def _(i=i):
                rdmas[i].wait_send()

    hbm = pltpu.MemorySpace.HBM
    return pl.pallas_call(
        body,
        out_shape=jax.ShapeDtypeStruct((b, sq_loc, d_model), jnp.bfloat16),
        in_specs=[pl.BlockSpec(memory_space=hbm)] * 5,
        out_specs=pl.BlockSpec(memory_space=hbm),
        scratch_shapes=[
            pltpu.VMEM((b, sq_loc, d_model), jnp.bfloat16),
            pltpu.VMEM((d_model, hq * dh), jnp.bfloat16),
            pltpu.VMEM((b, hq, dh, sq_loc), jnp.bfloat16),
            pltpu.VMEM((b, hq, dh, sq_loc), jnp.bfloat16),
            pltpu.VMEM((hq * dh, d_model), jnp.bfloat16),
            pltpu.VMEM((b, sq_loc, d_model), jnp.bfloat16),
            pltpu.VMEM((2, 2, b, hq, dh, sq_loc), jnp.bfloat16),
            pltpu.SemaphoreType.DMA((5,)),
            pltpu.SemaphoreType.DMA,
            pltpu.SemaphoreType.DMA((4,)),
            pltpu.SemaphoreType.DMA((4,)),
        ],
        compiler_params=pltpu.CompilerParams(collective_id=0),
    )(xb, wqb, kt, vt, wob)


# device time: 12277 ns/iter; 1.0459x vs baseline; 1.0459x over previous
import jax
import jax.numpy as jnp
from jax import lax
from jax.experimental import pallas as pl
from jax.experimental.pallas import tpu as pltpu

N_DEV = 16

_SCORE_DOT = (((2,), (1,)), ((0,), (0,)))
_CTX_DOT = (((2,), (2,)), ((0,), (0,)))
_OUT_DOT = (((0, 2), (0, 1)), ((), ()))


def kernel(x, Wq, K_ext, V_ext, Wo):
    b, sq_loc, d_model = x.shape
    _, _, hq, dh = K_ext.shape

    kt = jnp.transpose(K_ext, (0, 2, 3, 1)).astype(jnp.bfloat16)
    vt = jnp.transpose(V_ext, (0, 2, 3, 1)).astype(jnp.bfloat16)
    xb = x.astype(jnp.bfloat16)
    wqb = Wq.astype(jnp.bfloat16)
    wob = Wo.reshape(hq, dh, d_model).astype(jnp.bfloat16)

    def body(x_ref, wq_ref, kt_ref, vt_ref, wo_ref, out_ref,
             k_halo, v_halo, send_sems, recv_sems):
        my = lax.axis_index("i")
        left = lax.rem(my - 1 + N_DEV, N_DEV)
        right = lax.rem(my + 1, N_DEV)
        has_left = my != 0
        has_right = my != N_DEV - 1

        v_halo[:, :, :, dh, :] = jnp.ones(
            (2, b, hq, sq_loc), dtype=jnp.bfloat16)

        barrier_sem = pltpu.get_barrier_semaphore()

        @pl.when(has_right)
        def _():
            pl.semaphore_signal(
                barrier_sem, inc=1,
                device_id=(right,), device_id_type=pl.DeviceIdType.MESH,
            )

        @pl.when(has_left)
        def _():
            pl.semaphore_signal(
                barrier_sem, inc=1,
                device_id=(left,), device_id_type=pl.DeviceIdType.MESH,
            )

        is_edge = jnp.logical_or(my == 0, my == N_DEV - 1)

        @pl.when(is_edge)
        def _():
            pl.semaphore_wait(barrier_sem, 1)

        @pl.when(jnp.logical_not(is_edge))
        def _():
            pl.semaphore_wait(barrier_sem, 2)

        rdmas = []
        for i, (src, dst, tgt, gate) in enumerate([
            (kt_ref, k_halo.at[0], right, has_right),
            (vt_ref, v_halo.at[0, :, :, pl.ds(0, dh), :], right, has_right),
            (kt_ref, k_halo.at[1], left, has_left),
            (vt_ref, v_halo.at[1, :, :, pl.ds(0, dh), :], left, has_left),
        ]):
            c = pltpu.make_async_remote_copy(
                src_ref=src, dst_ref=dst,
                send_sem=send_sems.at[i], recv_sem=recv_sems.at[i],
                device_id=(tgt,), device_id_type=pl.DeviceIdType.MESH,
            )

            @pl.when(gate)
            def _(c=c):
                c.start()

            rdmas.append(c)

        xf = x_ref[...].reshape(b * sq_loc, d_model)
        qp = jnp.dot(xf, wq_ref[...], preferred_element_type=jnp.float32)
        q = (qp.reshape(b, sq_loc, hq, dh)
             .transpose(0, 2, 1, 3).astype(jnp.bfloat16))

        cd_acc = []
        for bb in range(b):
            s = lax.dot_general(q[bb], kt_ref[bb], _SCORE_DOT,
                                preferred_element_type=jnp.float32)
            w = jnp.exp(s * 0.125)
            ctx = lax.dot_general(w.astype(jnp.bfloat16), vt_ref[bb],
                                  _CTX_DOT, preferred_element_type=jnp.float32)
            den = jnp.sum(w, axis=-1, keepdims=True)
            cd_acc.append(jnp.concatenate([ctx, den], axis=-1))

        qi = lax.broadcasted_iota(jnp.int32, (1, sq_loc, sq_loc), 1)
        kj = lax.broadcasted_iota(jnp.int32, (1, sq_loc, sq_loc), 2)
        for slot, gate, mask in ((0, has_left, qi <= kj),
                                 (1, has_right, kj <= qi)):

            @pl.when(gate)
            def _(slot=slot):
                rdmas[2 * slot].wait_recv()

            ws = []
            for bb in range(b):
                s = lax.dot_general(q[bb], k_halo[slot, bb], _SCORE_DOT,
                                    preferred_element_type=jnp.float32)
                ws.append(jnp.where(mask, jnp.exp(s * 0.125), 0.0))

            @pl.when(gate)
            def _(slot=slot):
                rdmas[2 * slot + 1].wait_recv()

            for bb in range(b):
                cd_add = lax.dot_general(
                    ws[bb].astype(jnp.bfloat16), v_halo[slot, bb],
                    _CTX_DOT, preferred_element_type=jnp.float32)
                cd_acc[bb] += jnp.where(gate, cd_add, 0.0)

        ctx = jnp.stack([
            (cd_acc[bb][:, :, :dh] / cd_acc[bb][:, :, dh:]).transpose(1, 0, 2)
            for bb in range(b)
        ]).reshape(b * sq_loc, hq * dh)
        o = jnp.dot(ctx.astype(jnp.bfloat16),
                    wo_ref[...].reshape(hq * dh, d_model),
                    preferred_element_type=jnp.float32)
        out_ref[...] = o.reshape(b, sq_loc, d_model).astype(jnp.bfloat16)

        for i, gate in enumerate([has_right, has_right, has_left, has_left]):

            @pl.when(gate)
            def _(i=i):
                rdmas[i].wait_send()

    return pl.pallas_call(
        body,
        out_shape=jax.ShapeDtypeStruct((b, sq_loc, d_model), jnp.bfloat16),
        in_specs=[pl.BlockSpec(memory_space=pltpu.VMEM)] * 5,
        out_specs=pl.BlockSpec(memory_space=pltpu.VMEM),
        scratch_shapes=[
            pltpu.VMEM((2, b, hq, dh, sq_loc), jnp.bfloat16),
            pltpu.VMEM((2, b, hq, dh + 1, sq_loc), jnp.bfloat16),
            pltpu.SemaphoreType.DMA((4,)),
            pltpu.SemaphoreType.DMA((4,)),
        ],
        compiler_params=pltpu.CompilerParams(collective_id=0),
    )(xb, wqb, kt, vt, wob)
